# baseline (speedup 1.0000x reference)
import functools

import jax
import jax.numpy as jnp
from jax import lax
from jax.experimental import pallas as pl
from jax.experimental.pallas import tpu as pltpu, tpu_sc as plsc

NC = 2
NS = 16
NW = NC * NS

MAXLEN = 200
EMBED = 64
CB = 128


def _tok_pos_kernel(x_hbm, pos_hbm, tok_hbm, out_hbm,
                    idx_v, gbuf0, gbuf1, sbuf0, sbuf1, pos_v,
                    gsem0, gsem1, osem0, osem1):
    wid = lax.axis_index("s") * NC + lax.axis_index("c")

    pltpu.sync_copy(x_hbm.at[:, wid], idx_v)
    pltpu.sync_copy(pos_hbm, pos_v)

    gbufs = (gbuf0, gbuf1)
    sbufs = (sbuf0, sbuf1)
    gsem = (gsem0, gsem1)
    osem = (osem0, osem1)

    lane_rows = [lax.iota(jnp.int32, 16) + 16 * g for g in range(CB // 16)]

    def fire_gather(l, buf, sem):
        return pltpu.async_copy(
            tok_hbm.at[idx_v.at[l // 8, l % 8]], buf, sem)

    def transpose_add_flush(l, gbuf, sbuf, sem):
        def col_group(eo, _):
            pv = pos_v[l, pl.ds(eo * 16, 16)]
            for ei in range(16):
                e = eo * 16 + ei
                ev = jnp.full((16,), e, jnp.int32)
                pvec = jnp.full((16,), pv[ei], jnp.float32)
                for g in range(CB // 16):
                    vals = plsc.load_gather(gbuf, [lane_rows[g], ev])
                    sbuf[e // 8, e % 8, pl.ds(g * 16, 16)] = vals + pvec
            return ()
        lax.fori_loop(0, EMBED // 16, col_group, ())
        pltpu.async_copy(sbuf, out_hbm.at[l, :, wid], sem)

    def half(cc, par):
        l = cc * 2 + par

        @pl.when(cc >= 1)
        def _():
            pltpu.make_async_copy(
                sbufs[par], out_hbm.at[0, :, wid], osem[par]).wait()

        g = fire_gather(l, gbufs[par], gsem[par])

        @pl.when(l >= 1)
        def _():
            transpose_add_flush(l - 1, gbufs[1 - par], sbufs[1 - par],
                                osem[1 - par])

        g.wait()

    def body(cc, _):
        half(cc, 0)
        half(cc, 1)
        return ()

    lax.fori_loop(0, MAXLEN // 2, body, ())

    last = MAXLEN - 1
    transpose_add_flush(last, gbufs[last % 2], sbufs[last % 2], osem[last % 2])
    for par in range(2):
        pltpu.make_async_copy(
            sbufs[par], out_hbm.at[0, :, wid], osem[par]).wait()


def kernel(x, tok_table, pos_table):
    B, L = x.shape
    V, E = tok_table.shape
    assert E == EMBED and L == MAXLEN and B == NW * CB

    x4 = x.T.reshape(L // 8, 8, B // 128, 128).transpose(0, 2, 1, 3)

    fn = pl.kernel(
        _tok_pos_kernel,
        out_type=jax.ShapeDtypeStruct((L, E // 8, B // 128, 8, 128),
                                      jnp.float32),
        mesh=plsc.VectorSubcoreMesh(core_axis_name="c", subcore_axis_name="s"),
        scratch_types=[
            pltpu.VMEM((L // 8, 8, CB), jnp.int32),
            pltpu.VMEM((CB, EMBED), jnp.float32),
            pltpu.VMEM((CB, EMBED), jnp.float32),
            pltpu.VMEM((E // 8, 8, CB), jnp.float32),
            pltpu.VMEM((E // 8, 8, CB), jnp.float32),
            pltpu.VMEM((L, EMBED), jnp.float32),
            pltpu.SemaphoreType.DMA,
            pltpu.SemaphoreType.DMA,
            pltpu.SemaphoreType.DMA,
            pltpu.SemaphoreType.DMA,
        ],
        compiler_params=pltpu.CompilerParams(use_tc_tiling_on_sc=False,
                                             needs_layout_passes=False),
    )
    out5 = fn(x4, pos_table, tok_table)
    return out5.transpose(2, 4, 0, 1, 3).reshape(B, L, E)


# speedup vs baseline: 1.1298x; 1.1298x over previous
import functools

import jax
import jax.numpy as jnp
from jax import lax
from jax.experimental import pallas as pl
from jax.experimental.pallas import tpu as pltpu, tpu_sc as plsc

NC = 2
NS = 16
NW = NC * NS

MAXLEN = 200
EMBED = 64
CB = 128


def _tok_pos_kernel(x_hbm, pos_hbm, tok_hbm, out_hbm,
                    idx_v, gbuf0, gbuf1, sbuf0, sbuf1, pos_v,
                    gsem0, gsem1, osem0, osem1):
    wid = lax.axis_index("s") * NC + lax.axis_index("c")

    pltpu.sync_copy(x_hbm.at[:, wid], idx_v)
    pltpu.sync_copy(pos_hbm, pos_v)

    gbufs = (gbuf0, gbuf1)
    sbufs = (sbuf0, sbuf1)
    gsem = (gsem0, gsem1)
    osem = (osem0, osem1)

    ehi = [lax.iota(jnp.int32, 16) // 8 + 2 * j for j in range(EMBED // 16)]
    elo = [lax.iota(jnp.int32, 16) % 8 for j in range(EMBED // 16)]

    def fire_gather(l, buf, sem):
        return pltpu.async_copy(
            tok_hbm.at[idx_v.at[l // 8, l % 8]], buf, sem)

    def transpose_add_flush(l, gbuf, sbuf, sem):
        pvecs = [pos_v[l, pl.ds(j * 16, 16)] for j in range(EMBED // 16)]
        def row(t, _):
            tv = jnp.full((16,), t, jnp.int32)
            for j in range(EMBED // 16):
                vals = gbuf[t, pl.ds(j * 16, 16)] + pvecs[j]
                plsc.store_scatter(sbuf, [ehi[j], elo[j], tv], vals)
            return ()
        lax.fori_loop(0, CB, row, (), unroll=4)
        pltpu.async_copy(sbuf, out_hbm.at[l, :, wid], sem)

    def half(cc, par):
        l = cc * 2 + par

        @pl.when(cc >= 1)
        def _():
            pltpu.make_async_copy(
                sbufs[par], out_hbm.at[0, :, wid], osem[par]).wait()

        g = fire_gather(l, gbufs[par], gsem[par])

        @pl.when(l >= 1)
        def _():
            transpose_add_flush(l - 1, gbufs[1 - par], sbufs[1 - par],
                                osem[1 - par])

        g.wait()

    def body(cc, _):
        half(cc, 0)
        half(cc, 1)
        return ()

    lax.fori_loop(0, MAXLEN // 2, body, ())

    last = MAXLEN - 1
    transpose_add_flush(last, gbufs[last % 2], sbufs[last % 2], osem[last % 2])
    for par in range(2):
        pltpu.make_async_copy(
            sbufs[par], out_hbm.at[0, :, wid], osem[par]).wait()


def kernel(x, tok_table, pos_table):
    B, L = x.shape
    V, E = tok_table.shape
    assert E == EMBED and L == MAXLEN and B == NW * CB

    x4 = x.T.reshape(L // 8, 8, B // 128, 128).transpose(0, 2, 1, 3)

    fn = pl.kernel(
        _tok_pos_kernel,
        out_type=jax.ShapeDtypeStruct((L, E // 8, B // 128, 8, 128),
                                      jnp.float32),
        mesh=plsc.VectorSubcoreMesh(core_axis_name="c", subcore_axis_name="s"),
        scratch_types=[
            pltpu.VMEM((L // 8, 8, CB), jnp.int32),
            pltpu.VMEM((CB, EMBED), jnp.float32),
            pltpu.VMEM((CB, EMBED), jnp.float32),
            pltpu.VMEM((E // 8, 8, CB), jnp.float32),
            pltpu.VMEM((E // 8, 8, CB), jnp.float32),
            pltpu.VMEM((L, EMBED), jnp.float32),
            pltpu.SemaphoreType.DMA,
            pltpu.SemaphoreType.DMA,
            pltpu.SemaphoreType.DMA,
            pltpu.SemaphoreType.DMA,
        ],
        compiler_params=pltpu.CompilerParams(use_tc_tiling_on_sc=False,
                                             needs_layout_passes=False),
    )
    out5 = fn(x4, pos_table, tok_table)
    return out5.transpose(2, 4, 0, 1, 3).reshape(B, L, E)


# speedup vs baseline: 1.4634x; 1.2953x over previous
import functools

import jax
import jax.numpy as jnp
from jax import lax
from jax.experimental import pallas as pl
from jax.experimental.pallas import tpu as pltpu, tpu_sc as plsc

NC = 2
NS = 16
NW = NC * NS

MAXLEN = 200
EMBED = 64
CB = 128


def _tok_pos_kernel(x_hbm, pos_hbm, tok_hbm, out_hbm,
                    idx_v, gbuf0, gbuf1, sbuf0, sbuf1, pos_v,
                    gsem0, gsem1, osem0, osem1):
    wid = lax.axis_index("s") * NC + lax.axis_index("c")

    pltpu.sync_copy(x_hbm.at[:, wid], idx_v)
    pltpu.sync_copy(pos_hbm, pos_v)

    gbufs = (gbuf0, gbuf1)
    sbufs = (sbuf0, sbuf1)
    gsem = (gsem0, gsem1)
    osem = (osem0, osem1)

    ehi = [lax.iota(jnp.int32, 16) // 8 + 2 * j for j in range(EMBED // 16)]
    elo = [lax.iota(jnp.int32, 16) % 8 for j in range(EMBED // 16)]

    def fire_gather(l, buf, sem):
        return pltpu.async_copy(
            tok_hbm.at[idx_v.at[l // 8, l % 8]], buf, sem)

    def transpose_add_flush(l, gbuf, sbuf, sem):
        pvecs = [pos_v[l, pl.ds(j * 16, 16)] for j in range(EMBED // 16)]
        @plsc.parallel_loop(0, CB, unroll=4)
        def row(t):
            tv = jnp.full((16,), t, jnp.int32)
            for j in range(EMBED // 16):
                vals = gbuf[t, pl.ds(j * 16, 16)] + pvecs[j]
                plsc.store_scatter(sbuf, [ehi[j], elo[j], tv], vals)
        pltpu.async_copy(sbuf, out_hbm.at[l, :, wid], sem)

    def half(cc, par):
        l = cc * 2 + par

        @pl.when(cc >= 1)
        def _():
            pltpu.make_async_copy(
                sbufs[par], out_hbm.at[0, :, wid], osem[par]).wait()

        g = fire_gather(l, gbufs[par], gsem[par])

        @pl.when(l >= 1)
        def _():
            transpose_add_flush(l - 1, gbufs[1 - par], sbufs[1 - par],
                                osem[1 - par])

        g.wait()

    def body(cc, _):
        half(cc, 0)
        half(cc, 1)
        return ()

    lax.fori_loop(0, MAXLEN // 2, body, ())

    last = MAXLEN - 1
    transpose_add_flush(last, gbufs[last % 2], sbufs[last % 2], osem[last % 2])
    for par in range(2):
        pltpu.make_async_copy(
            sbufs[par], out_hbm.at[0, :, wid], osem[par]).wait()


def kernel(x, tok_table, pos_table):
    B, L = x.shape
    V, E = tok_table.shape
    assert E == EMBED and L == MAXLEN and B == NW * CB

    x4 = x.T.reshape(L // 8, 8, B // 128, 128).transpose(0, 2, 1, 3)

    fn = pl.kernel(
        _tok_pos_kernel,
        out_type=jax.ShapeDtypeStruct((L, E // 8, B // 128, 8, 128),
                                      jnp.float32),
        mesh=plsc.VectorSubcoreMesh(core_axis_name="c", subcore_axis_name="s"),
        scratch_types=[
            pltpu.VMEM((L // 8, 8, CB), jnp.int32),
            pltpu.VMEM((CB, EMBED), jnp.float32),
            pltpu.VMEM((CB, EMBED), jnp.float32),
            pltpu.VMEM((E // 8, 8, CB), jnp.float32),
            pltpu.VMEM((E // 8, 8, CB), jnp.float32),
            pltpu.VMEM((L, EMBED), jnp.float32),
            pltpu.SemaphoreType.DMA,
            pltpu.SemaphoreType.DMA,
            pltpu.SemaphoreType.DMA,
            pltpu.SemaphoreType.DMA,
        ],
        compiler_params=pltpu.CompilerParams(use_tc_tiling_on_sc=False,
                                             needs_layout_passes=False),
    )
    out5 = fn(x4, pos_table, tok_table)
    return out5.transpose(2, 4, 0, 1, 3).reshape(B, L, E)
